# Initial kernel scaffold; baseline (speedup 1.0000x reference)
import jax
import jax.numpy as jnp
from jax.experimental import pallas as pl


def kernel(word_idx, emb):
    raise NotImplementedError("write your pallas kernel here")


# speedup vs baseline: 1.8090x; 1.8090x over previous
import jax
import jax.numpy as jnp
from jax import lax
from jax.experimental import pallas as pl
from jax.experimental.pallas import tpu as pltpu
from jax.experimental.pallas import tpu_sc as plsc

N_ROWS = 16384 * 50
D = 64
NC, NS = 2, 16
NW = NC * NS
B_PER_W = N_ROWS // NW
CHUNK = 512
N_CHUNKS = B_PER_W // CHUNK


def _gather_body(idx_hbm, table_hbm, out_hbm, idx_v, rows_v, sem):
    wid = lax.axis_index("s") * NC + lax.axis_index("c")
    base = wid * B_PER_W

    def body(j, carry):
        rb = base + j * CHUNK
        pltpu.sync_copy(idx_hbm.at[pl.ds(rb, CHUNK)], idx_v)
        pltpu.async_copy(table_hbm.at[idx_v], rows_v, sem).wait()
        pltpu.sync_copy(rows_v, out_hbm.at[pl.ds(rb, CHUNK)])
        return carry

    lax.fori_loop(0, N_CHUNKS, body, 0)


def kernel(word_idx, emb):
    idx = word_idx.reshape(-1)
    mesh = plsc.VectorSubcoreMesh(core_axis_name="c", subcore_axis_name="s")
    f = pl.kernel(
        _gather_body,
        out_type=jax.ShapeDtypeStruct((N_ROWS, D), jnp.float32),
        mesh=mesh,
        scratch_types=[
            pltpu.VMEM((CHUNK,), jnp.int32),
            pltpu.VMEM((CHUNK, D), jnp.float32),
            pltpu.SemaphoreType.DMA,
        ],
        compiler_params=pltpu.CompilerParams(use_tc_tiling_on_sc=False),
    )
    out = f(idx, emb)
    return out.reshape(word_idx.shape[0], word_idx.shape[1], D)


# speedup vs baseline: 1.8737x; 1.0357x over previous
import jax
import jax.numpy as jnp
from jax import lax
from jax.experimental import pallas as pl
from jax.experimental.pallas import tpu as pltpu
from jax.experimental.pallas import tpu_sc as plsc

N_ROWS = 16384 * 50
D = 64
NC, NS = 2, 16
NW = NC * NS
B_PER_W = N_ROWS // NW
CHUNK = 512
NBUF = 2
N_CHUNKS = B_PER_W // CHUNK
N_GROUPS = N_CHUNKS // NBUF
assert B_PER_W % (CHUNK * NBUF) == 0


def _gather_body(idx_hbm, table_hbm, out_hbm, idx_v, rows_v, isems, gsems, osems):
    wid = lax.axis_index("s") * NC + lax.axis_index("c")
    base = wid * B_PER_W

    def issue_idx(j, b):
        pltpu.async_copy(
            idx_hbm.at[pl.ds(base + j * CHUNK, CHUNK)], idx_v.at[b], isems[b])

    def wait_idx(b):
        pltpu.make_async_copy(
            idx_hbm.at[pl.ds(base, CHUNK)], idx_v.at[b], isems[b]).wait()

    def issue_gather(b):
        pltpu.async_copy(table_hbm.at[idx_v.at[b]], rows_v.at[b], gsems[b])

    def wait_gather(b):
        pltpu.make_async_copy(
            table_hbm.at[idx_v.at[b]], rows_v.at[b], gsems[b]).wait()

    def issue_out(j, b):
        pltpu.async_copy(
            rows_v.at[b], out_hbm.at[pl.ds(base + j * CHUNK, CHUNK)], osems[b])

    def wait_out(b):
        pltpu.make_async_copy(
            rows_v.at[b], out_hbm.at[pl.ds(base, CHUNK)], osems[b]).wait()

    def finalize(k, b, last):
        wait_gather(b)
        issue_out(k, b)
        if not last:
            issue_idx(jnp.minimum(k + NBUF, N_CHUNKS - 1), b)

    for b in range(NBUF):
        issue_idx(b, b)
    for b in range(NBUF):
        wait_idx(b)
        issue_gather(b)
        if b > 0:
            finalize(b - 1, b - 1, last=False)

    @pl.loop(1, N_GROUPS)
    def _group(g):
        j0 = g * NBUF
        for b in range(NBUF):
            j = j0 + b
            wait_idx(b)
            wait_out(b)
            issue_gather(b)
            pb = (b - 1) % NBUF
            finalize(j - 1, pb, last=False)

    last_b = (N_CHUNKS - 1) % NBUF
    finalize(N_CHUNKS - 1, last_b, last=True)
    for b in range(NBUF):
        wait_out(b)
    for b in range(NBUF):
        if b != last_b:
            wait_idx(b)


def kernel(word_idx, emb):
    idx = word_idx.reshape(-1)
    mesh = plsc.VectorSubcoreMesh(core_axis_name="c", subcore_axis_name="s")
    f = pl.kernel(
        _gather_body,
        out_type=jax.ShapeDtypeStruct((N_ROWS, D), jnp.float32),
        mesh=mesh,
        scratch_types=[
            pltpu.VMEM((NBUF, CHUNK), jnp.int32),
            pltpu.VMEM((NBUF, CHUNK, D), jnp.float32),
            [pltpu.SemaphoreType.DMA] * NBUF,
            [pltpu.SemaphoreType.DMA] * NBUF,
            [pltpu.SemaphoreType.DMA] * NBUF,
        ],
        compiler_params=pltpu.CompilerParams(use_tc_tiling_on_sc=False),
    )
    out = f(idx, emb)
    return out.reshape(word_idx.shape[0], word_idx.shape[1], D)
